# baseline (speedup 1.0000x reference)
import functools

import jax
import jax.numpy as jnp
from jax import lax
from jax.experimental import pallas as pl
from jax.experimental.pallas import tpu as pltpu
from jax.experimental.pallas import tpu_sc as plsc

_D = 1024
_SCALE = float(_D ** 0.5)
_LANES = 16

_NC = 2
_NS = 16
_NW = _NC * _NS
_NR = 4
_NSRC = 3
_PREF = 3


def _sc_body(tok_per_w, chunk, idx_hbm, src_hbm, table_hbm, out_hbm,
             idx_v, rows_v, src_v, gsem, ssem, osem):
    wid = lax.axis_index("s") * _NC + lax.axis_index("c")
    base = wid * tok_per_w
    n_chunks = tok_per_w // chunk

    pltpu.sync_copy(idx_hbm.at[pl.ds(base, tok_per_w)], idx_v)

    def issue_in(c):
        rb, sb = c % _NR, c % _NSRC
        coff = pl.multiple_of(c * chunk, 8)
        pltpu.async_copy(table_hbm.at[idx_v.at[pl.ds(coff, chunk)]],
                         rows_v.at[rb], gsem.at[rb])
        pltpu.async_copy(src_hbm.at[pl.ds(base + coff, chunk)],
                         src_v.at[sb], ssem.at[sb])

    def wait_sem(sem_slice, dst):
        pltpu.make_async_copy(src_hbm.at[pl.ds(0, chunk)], dst,
                              sem_slice).wait()

    for c in range(_PREF):
        issue_in(c)

    def chunk_body(c, carry):
        rb, sb = c % _NR, c % _NSRC
        coff = pl.multiple_of(c * chunk, 8)
        wait_sem(gsem.at[rb], rows_v.at[rb])
        wait_sem(ssem.at[sb], src_v.at[sb])

        def row_body(r, rcarry):
            for j in range(_D // _LANES):
                sl = pl.ds(j * _LANES, _LANES)
                plsc.addupdate(rows_v.at[rb, r, sl],
                               src_v[sb, r, sl] * _SCALE)
            return rcarry

        lax.fori_loop(0, chunk, row_body, 0)

        pltpu.async_copy(rows_v.at[rb],
                         out_hbm.at[pl.ds(base + coff, chunk)], osem.at[rb])

        @pl.when(c + _PREF < n_chunks)
        def _():
            @pl.when(c >= 1)
            def _():
                wait_sem(osem.at[(c - 1) % _NR], rows_v.at[(c - 1) % _NR])

            issue_in(c + _PREF)

        return carry

    lax.fori_loop(0, n_chunks, chunk_body, 0)

    for b in range(_NR):
        wait_sem(osem.at[b], rows_v.at[b])


@functools.partial(jax.jit, static_argnames=("n_tok", "chunk"))
def _sc_call(idx, src, table, n_tok, chunk):
    tok_per_w = n_tok // _NW
    mesh = plsc.VectorSubcoreMesh(core_axis_name="c", subcore_axis_name="s")
    body = functools.partial(_sc_body, tok_per_w, chunk)
    return pl.kernel(
        body,
        out_type=jax.ShapeDtypeStruct((n_tok, _D), jnp.float32),
        mesh=mesh,
        scratch_types=[
            pltpu.VMEM((tok_per_w,), jnp.int32),
            pltpu.VMEM((_NR, chunk, _D), jnp.float32),
            pltpu.VMEM((_NSRC, chunk, _D), jnp.float32),
            pltpu.SemaphoreType.DMA((_NR,)),
            pltpu.SemaphoreType.DMA((_NSRC,)),
            pltpu.SemaphoreType.DMA((_NR,)),
        ],
    )(idx, src, table)


def kernel(src_word, src_pos, pos_table):
    b, s, d = src_word.shape
    n_tok = b * s
    src = src_word.reshape(n_tok, d)
    idx = src_pos.reshape(n_tok)
    out = _sc_call(idx, src, pos_table, n_tok, 16)
    return out.reshape(b, s, d)
